# Initial kernel scaffold; baseline (speedup 1.0000x reference)
import functools

import jax
import jax.numpy as jnp
from jax import lax
from jax.experimental import pallas as pl
from jax.experimental.pallas import tpu as pltpu
from jax.experimental.pallas import tpu_sc as plsc

_N = 10000
_E = 320000
_NC = 2
_NS = 16
_NW = _NC * _NS
_B = 128
_EPT = 10240
_NB = _EPT // _B
_EPAD = _NW * _EPT
_NPT = _N // _NS

_mesh = lambda: plsc.VectorSubcoreMesh(core_axis_name="c", subcore_axis_name="s")


@functools.partial(
    pl.kernel,
    out_type=jax.ShapeDtypeStruct((_NC, _N, 1), jnp.float32),
    mesh=_mesh(),
    scratch_types=[
        pltpu.VMEM((_NB, _B), jnp.int32),
        pltpu.VMEM((_EPT, 1), jnp.float32),
        pltpu.VMEM_SHARED((_N, 1), jnp.float32),
    ],
)
def _deg_sc(src_hbm, w_hbm, z_hbm, out_hbm, src_v, w_v, acc):
    c = lax.axis_index("c")
    s = lax.axis_index("s")
    wid = s * _NC + c
    pltpu.sync_copy(src_hbm.at[wid], src_v)
    pltpu.sync_copy(w_hbm.at[pl.ds(wid * _EPT, _EPT)], w_v)
    pltpu.sync_copy(z_hbm.at[pl.ds(s * _NPT, _NPT)], acc.at[pl.ds(s * _NPT, _NPT)])
    plsc.subcore_barrier()

    def body(j, carry):
        pltpu.sync_copy(w_v.at[pl.ds(j * _B, _B)], acc.at[src_v.at[j]], add=True)
        return carry

    lax.fori_loop(0, _NB, body, 0)
    plsc.subcore_barrier()
    pltpu.sync_copy(acc.at[pl.ds(s * _NPT, _NPT)],
                    out_hbm.at[c, pl.ds(s * _NPT, _NPT)])


def _make_prop(F):
    nf = F // 16

    @functools.partial(
        pl.kernel,
        out_type=jax.ShapeDtypeStruct((_NC, _N, F), jnp.float32),
        mesh=_mesh(),
        scratch_types=[
            pltpu.VMEM((_N,), jnp.float32),
            pltpu.VMEM((_NB, _B), jnp.int32),
            pltpu.VMEM((_NB, _B), jnp.int32),
            pltpu.VMEM((_EPT,), jnp.float32),
            pltpu.VMEM((_B,), jnp.float32),
            pltpu.VMEM((_B, F), jnp.float32),
            pltpu.VMEM_SHARED((_N, F), jnp.float32),
            pltpu.SemaphoreType.DMA,
        ],
    )
    def _prop(h_hbm, dinv_hbm, src_hbm, dst_hbm, w_hbm, z_hbm, out_hbm,
              dinv_v, src_v, dst_v, w_v, norm_v, rows_v, acc, sem):
        c = lax.axis_index("c")
        s = lax.axis_index("s")
        wid = s * _NC + c
        pltpu.sync_copy(dinv_hbm, dinv_v)
        pltpu.sync_copy(src_hbm.at[wid], src_v)
        pltpu.sync_copy(dst_hbm.at[wid], dst_v)
        pltpu.sync_copy(w_hbm.at[pl.ds(wid * _EPT, _EPT)], w_v)
        pltpu.sync_copy(z_hbm.at[pl.ds(s * _NPT, _NPT)],
                        acc.at[pl.ds(s * _NPT, _NPT)])
        plsc.subcore_barrier()

        def batch(j, carry):
            cp = pltpu.async_copy(h_hbm.at[src_v.at[j]], rows_v, sem)
            for g in range(_B // 16):
                sl16 = pl.ds(g * 16, 16)
                s16 = src_v[j, sl16]
                d16 = dst_v[j, sl16]
                w16 = w_v[pl.ds(j * _B + g * 16, 16)]
                da = plsc.load_gather(dinv_v, [s16])
                db = plsc.load_gather(dinv_v, [d16])
                norm_v[sl16] = -(da * w16 * db)
            cp.wait()

            def scale(i, carry2):
                nv = plsc.load_gather(norm_v, [jnp.full((16,), i, jnp.int32)])
                for f in range(nf):
                    slf = pl.ds(f * 16, 16)
                    rows_v[i, slf] = rows_v[i, slf] * nv
                return carry2

            lax.fori_loop(0, _B, scale, 0)
            pltpu.sync_copy(rows_v, acc.at[dst_v.at[j]], add=True)
            return carry

        lax.fori_loop(0, _NB, batch, 0)
        plsc.subcore_barrier()
        pltpu.sync_copy(acc.at[pl.ds(s * _NPT, _NPT)],
                        out_hbm.at[c, pl.ds(s * _NPT, _NPT)])

    return _prop


_prop64 = _make_prop(64)
_prop32 = _make_prop(32)


_P = lax.Precision.HIGHEST


def _tc_a_body(x_ref, w_ref, b_ref, degp_ref, h_ref, dinv_ref):
    deg = degp_ref[0] + degp_ref[1]
    safe = jnp.where(deg > 0, deg, 1.0)
    dinv_ref[...] = jnp.where(deg > 0, lax.rsqrt(safe), 0.0)
    h = jnp.dot(x_ref[...], w_ref[...], precision=_P) + b_ref[...][None, :]
    h_ref[...] = jnp.maximum(h, 0.0)


def _tc_b_body(p_ref, h_ref, cw_ref, tx1_ref, acc_ref):
    tx1 = p_ref[0] + p_ref[1]
    tx1_ref[...] = tx1
    acc_ref[...] = (jnp.dot(h_ref[...], cw_ref[0] - cw_ref[2], precision=_P)
                    + jnp.dot(tx1, cw_ref[1], precision=_P))


def _tc_c_body(q_ref, acc_ref, cw_ref, cb_ref, out_ref):
    tx2p = q_ref[0] + q_ref[1]
    out = acc_ref[...] + 2.0 * jnp.dot(tx2p, cw_ref[2], precision=_P)
    out_ref[...] = jnp.maximum(out + cb_ref[...][None, :], 0.0)


def _tc_head_body(q_ref, acc_ref, cw_ref, cb_ref, w2_ref, b2_ref, out_ref):
    tx2p = q_ref[0] + q_ref[1]
    h2 = acc_ref[...] + 2.0 * jnp.dot(tx2p, cw_ref[2], precision=_P)
    h2 = jnp.maximum(h2 + cb_ref[...][None, :], 0.0)
    logits = jnp.dot(h2, w2_ref[...], precision=_P) + b2_ref[...][None, :]
    m = jnp.max(logits, axis=1, keepdims=True)
    e = jnp.exp(logits - m)
    out_ref[...] = e / jnp.sum(e, axis=1, keepdims=True)


def _call(body, out_shapes, *args):
    return pl.pallas_call(body, out_shape=out_shapes)(*args)


def kernel(x, edge_index, edge_weight, W1, b1, c1_w, c1_b, c2_w, c2_b, W2, b2):
    pad = _EPAD - _E
    src = jnp.concatenate([edge_index[0], jnp.zeros((pad,), jnp.int32)])
    dst = jnp.concatenate([edge_index[1], jnp.zeros((pad,), jnp.int32)])
    w = jnp.concatenate([edge_weight, jnp.zeros((pad,), jnp.float32)])
    src2d = src.reshape(_NW, _NB, _B)
    dst2d = dst.reshape(_NW, _NB, _B)
    wcol = w.reshape(_EPAD, 1)
    z1 = jnp.zeros((_N, 1), jnp.float32)
    z64 = jnp.zeros((_N, 64), jnp.float32)
    z32 = jnp.zeros((_N, 32), jnp.float32)

    degp = _deg_sc(src2d, wcol, z1)
    h0, dinv1 = _call(
        _tc_a_body,
        (jax.ShapeDtypeStruct((_N, 64), jnp.float32),
         jax.ShapeDtypeStruct((_N, 1), jnp.float32)),
        x, W1, b1, degp)
    dinv = dinv1.reshape(_N)

    p1 = _prop64(h0, dinv, src2d, dst2d, w, z64)
    tx1, acc1 = _call(
        _tc_b_body,
        (jax.ShapeDtypeStruct((_N, 64), jnp.float32),
         jax.ShapeDtypeStruct((_N, 32), jnp.float32)),
        p1, h0, c1_w)
    q1 = _prop64(tx1, dinv, src2d, dst2d, w, z64)
    h1 = _call(_tc_c_body, jax.ShapeDtypeStruct((_N, 32), jnp.float32),
               q1, acc1, c1_w, c1_b)

    p2 = _prop32(h1, dinv, src2d, dst2d, w, z32)
    tx1b, acc2 = _call(
        _tc_b_body,
        (jax.ShapeDtypeStruct((_N, 32), jnp.float32),
         jax.ShapeDtypeStruct((_N, 16), jnp.float32)),
        p2, h1, c2_w)
    q2 = _prop32(tx1b, dinv, src2d, dst2d, w, z32)
    probs = _call(_tc_head_body, jax.ShapeDtypeStruct((_N, 2), jnp.float32),
                  q2, acc2, c2_w, c2_b, W2, b2)
    return probs


# speedup vs baseline: 6.8553x; 6.8553x over previous
import functools

import jax
import jax.numpy as jnp
from jax import lax
from jax.experimental import pallas as pl
from jax.experimental.pallas import tpu as pltpu
from jax.experimental.pallas import tpu_sc as plsc

_N = 10000
_E = 320000
_NC = 2
_NS = 16
_NW = _NC * _NS
_B = 128
_EPT = 10240
_NB = _EPT // _B
_EPAD = _NW * _EPT
_NA = 10240
_NPT = _NA // _NS
_DR = 79
_ND = _DR * 128

_MESH = plsc.VectorSubcoreMesh(core_axis_name="c", subcore_axis_name="s")


@functools.partial(
    pl.kernel,
    out_type=jax.ShapeDtypeStruct((_NC, _NA, 16), jnp.float32),
    mesh=_MESH,
    compiler_params=pltpu.CompilerParams(needs_layout_passes=False, use_tc_tiling_on_sc=False),
    scratch_types=[
        pltpu.VMEM((_NB, _B), jnp.int32),
        pltpu.VMEM((_B, 16), jnp.float32),
        pltpu.VMEM_SHARED((_NA, 16), jnp.float32),
    ],
)
def _deg_sc(src_hbm, w_hbm, z_hbm, out_hbm, src_v, w_v, acc):
    c = lax.axis_index("c")
    s = lax.axis_index("s")
    wid = s * _NC + c
    pltpu.sync_copy(src_hbm.at[wid], src_v)
    pltpu.sync_copy(z_hbm.at[pl.ds(s * _NPT, _NPT)], acc.at[pl.ds(s * _NPT, _NPT)])
    plsc.subcore_barrier()

    def body(j, carry):
        pltpu.sync_copy(w_hbm.at[wid * _NB + j], w_v)
        pltpu.sync_copy(w_v, acc.at[src_v.at[j]], add=True)
        return carry

    lax.fori_loop(0, _NB, body, 0)
    plsc.subcore_barrier()
    pltpu.sync_copy(acc.at[pl.ds(s * _NPT, _NPT)],
                    out_hbm.at[c, pl.ds(s * _NPT, _NPT)])


def _make_prop(F):
    nf = F // 16

    @functools.partial(
        pl.kernel,
        out_type=jax.ShapeDtypeStruct((_NC, _NA, F), jnp.float32),
        mesh=_MESH,
        compiler_params=pltpu.CompilerParams(needs_layout_passes=False, use_tc_tiling_on_sc=False),
        scratch_types=[
            pltpu.VMEM((_DR, 128), jnp.float32),
            pltpu.VMEM((_NB, _B), jnp.int32),
            pltpu.VMEM((_NB, _B), jnp.int32),
            pltpu.VMEM((_NB, _B), jnp.float32),
            pltpu.VMEM((_B,), jnp.float32),
            pltpu.VMEM((_B, F), jnp.float32),
            pltpu.VMEM_SHARED((_NA, F), jnp.float32),
            pltpu.SemaphoreType.DMA,
        ],
    )
    def _prop(h_hbm, dinv_hbm, src_hbm, dst_hbm, w_hbm, z_hbm, out_hbm,
              dinv_v, src_v, dst_v, w_v, norm_v, rows_v, acc, sem):
        c = lax.axis_index("c")
        s = lax.axis_index("s")
        wid = s * _NC + c
        pltpu.sync_copy(dinv_hbm, dinv_v)
        pltpu.sync_copy(src_hbm.at[wid], src_v)
        pltpu.sync_copy(dst_hbm.at[wid], dst_v)
        pltpu.sync_copy(w_hbm.at[wid], w_v)
        pltpu.sync_copy(z_hbm.at[pl.ds(s * _NPT, _NPT)],
                        acc.at[pl.ds(s * _NPT, _NPT)])
        plsc.subcore_barrier()

        def batch(j, carry):
            cp = pltpu.async_copy(h_hbm.at[src_v.at[j]], rows_v, sem)
            for g in range(_B // 16):
                sl16 = pl.ds(g * 16, 16)
                s16 = src_v[j, sl16]
                d16 = dst_v[j, sl16]
                w16 = w_v[j, sl16]
                da = plsc.load_gather(
                    dinv_v,
                    [lax.shift_right_logical(s16, 7), lax.bitwise_and(s16, 127)])
                db = plsc.load_gather(
                    dinv_v,
                    [lax.shift_right_logical(d16, 7), lax.bitwise_and(d16, 127)])
                norm_v[sl16] = -(da * w16 * db)
            cp.wait()

            def scale(i, carry2):
                nv = plsc.load_gather(norm_v, [jnp.full((16,), i, jnp.int32)])
                for f in range(nf):
                    slf = pl.ds(f * 16, 16)
                    rows_v[i, slf] = rows_v[i, slf] * nv
                return carry2

            lax.fori_loop(0, _B, scale, 0)
            pltpu.sync_copy(rows_v, acc.at[dst_v.at[j]], add=True)
            return carry

        lax.fori_loop(0, _NB, batch, 0)
        plsc.subcore_barrier()
        pltpu.sync_copy(acc.at[pl.ds(s * _NPT, _NPT)],
                        out_hbm.at[c, pl.ds(s * _NPT, _NPT)])

    return _prop


_prop64 = _make_prop(64)
_prop32 = _make_prop(32)


_P = lax.Precision.HIGHEST
_RB = 2000


def _tc_a_body(x_ref, w_ref, b_ref, degp_ref, h_ref, dinv_ref):
    deg = degp_ref[0, :, 0:1] + degp_ref[1, :, 0:1]
    safe = jnp.where(deg > 0, deg, 1.0)
    dinv_ref[...] = jnp.where(deg > 0, lax.rsqrt(safe), 0.0)
    h = jnp.dot(x_ref[...], w_ref[...], precision=_P) + b_ref[...][None, :]
    h_ref[...] = jnp.maximum(h, 0.0)


def _tc_b_body(p_ref, h_ref, cw_ref, tx1_ref, acc_ref):
    tx1 = p_ref[0] + p_ref[1]
    tx1_ref[...] = tx1
    acc_ref[...] = (jnp.dot(h_ref[...], cw_ref[0] - cw_ref[2], precision=_P)
                    + jnp.dot(tx1, cw_ref[1], precision=_P))


def _tc_c_body(q_ref, acc_ref, cw_ref, cb_ref, out_ref):
    tx2p = q_ref[0] + q_ref[1]
    out = acc_ref[...] + 2.0 * jnp.dot(tx2p, cw_ref[2], precision=_P)
    out_ref[...] = jnp.maximum(out + cb_ref[...][None, :], 0.0)


def _tc_head_body(q_ref, acc_ref, cw_ref, cb_ref, w2_ref, b2_ref, out_ref):
    tx2p = q_ref[0] + q_ref[1]
    h2 = acc_ref[...] + 2.0 * jnp.dot(tx2p, cw_ref[2], precision=_P)
    h2 = jnp.maximum(h2 + cb_ref[...][None, :], 0.0)
    logits = jnp.dot(h2, w2_ref[...], precision=_P) + b2_ref[...][None, :]
    m = jnp.max(logits, axis=1, keepdims=True)
    e = jnp.exp(logits - m)
    out_ref[...] = e / jnp.sum(e, axis=1, keepdims=True)


def _rows(shape):
    if len(shape) == 3:
        return pl.BlockSpec((2, _RB, shape[2]), lambda i: (0, i, 0))
    return pl.BlockSpec((_RB,) + shape[1:], lambda i: (i,) + (0,) * (len(shape) - 1))


def _full(shape):
    return pl.BlockSpec(shape, lambda i: (0,) * len(shape))


def _call(body, out_shapes, specs, *args):
    grid = (_N // _RB,)
    in_specs = [s(a.shape) for s, a in zip(specs, args)]
    out_specs = jax.tree.map(lambda o: _rows(o.shape), out_shapes,
                             is_leaf=lambda o: isinstance(o, jax.ShapeDtypeStruct))
    return pl.pallas_call(body, grid=grid, in_specs=in_specs,
                          out_specs=out_specs, out_shape=out_shapes)(*args)


def kernel(x, edge_index, edge_weight, W1, b1, c1_w, c1_b, c2_w, c2_b, W2, b2):
    pad = _EPAD - _E
    src = jnp.concatenate([edge_index[0], jnp.zeros((pad,), jnp.int32)])
    dst = jnp.concatenate([edge_index[1], jnp.zeros((pad,), jnp.int32)])
    w = jnp.concatenate([edge_weight, jnp.zeros((pad,), jnp.float32)])
    src2d = src.reshape(_NW, _NB, _B)
    dst2d = dst.reshape(_NW, _NB, _B)
    w2d = w.reshape(_NW, _NB, _B)
    w16 = jnp.broadcast_to(w[:, None], (_EPAD, 16)).reshape(_NW * _NB, _B, 16)
    z16 = jnp.zeros((_NA, 16), jnp.float32)
    z64 = jnp.zeros((_NA, 64), jnp.float32)
    z32 = jnp.zeros((_NA, 32), jnp.float32)

    degp = _deg_sc(src2d, w16, z16)
    h0, dinv1 = _call(
        _tc_a_body,
        (jax.ShapeDtypeStruct((_N, 64), jnp.float32),
         jax.ShapeDtypeStruct((_N, 1), jnp.float32)),
        (_rows, _full, _full, _rows),
        x, W1, b1, degp)
    dinv = jnp.pad(dinv1.reshape(_N), (0, _ND - _N)).reshape(_DR, 128)

    p1 = _prop64(h0, dinv, src2d, dst2d, w2d, z64)
    tx1, acc1 = _call(
        _tc_b_body,
        (jax.ShapeDtypeStruct((_N, 64), jnp.float32),
         jax.ShapeDtypeStruct((_N, 32), jnp.float32)),
        (_rows, _rows, _full),
        p1, h0, c1_w)
    q1 = _prop64(tx1, dinv, src2d, dst2d, w2d, z64)
    h1 = _call(_tc_c_body, jax.ShapeDtypeStruct((_N, 32), jnp.float32),
               (_rows, _rows, _full, _full), q1, acc1, c1_w, c1_b)

    p2 = _prop32(h1, dinv, src2d, dst2d, w2d, z32)
    tx1b, acc2 = _call(
        _tc_b_body,
        (jax.ShapeDtypeStruct((_N, 32), jnp.float32),
         jax.ShapeDtypeStruct((_N, 16), jnp.float32)),
        (_rows, _rows, _full),
        p2, h1, c2_w)
    q2 = _prop32(tx1b, dinv, src2d, dst2d, w2d, z32)
    probs = _call(_tc_head_body, jax.ShapeDtypeStruct((_N, 2), jnp.float32),
                  (_rows, _rows, _full, _full, _full, _full),
                  q2, acc2, c2_w, c2_b, W2, b2)
    return probs


# speedup vs baseline: 8.4519x; 1.2329x over previous
import functools

import jax
import jax.numpy as jnp
from jax import lax
from jax.experimental import pallas as pl
from jax.experimental.pallas import tpu as pltpu
from jax.experimental.pallas import tpu_sc as plsc

_N = 10000
_E = 320000
_NC = 2
_NS = 16
_NW = _NC * _NS
_B = 128
_EPT = 10240
_NB = _EPT // _B
_EPAD = _NW * _EPT
_NA = 10240
_NPT = _NA // _NS
_DR = 79
_ND = _DR * 128

_MESH = plsc.VectorSubcoreMesh(core_axis_name="c", subcore_axis_name="s")


@functools.partial(
    pl.kernel,
    out_type=jax.ShapeDtypeStruct((_NC, _NA, 16), jnp.float32),
    mesh=_MESH,
    compiler_params=pltpu.CompilerParams(needs_layout_passes=False, use_tc_tiling_on_sc=False),
    scratch_types=[
        pltpu.VMEM((_NB, _B), jnp.int32),
        pltpu.VMEM((_B, 16), jnp.float32),
        pltpu.VMEM_SHARED((_NA, 16), jnp.float32),
    ],
)
def _deg_sc(src_hbm, w_hbm, z_hbm, out_hbm, src_v, w_v, acc):
    c = lax.axis_index("c")
    s = lax.axis_index("s")
    wid = s * _NC + c
    pltpu.sync_copy(src_hbm.at[wid], src_v)
    pltpu.sync_copy(z_hbm.at[pl.ds(s * _NPT, _NPT)], acc.at[pl.ds(s * _NPT, _NPT)])
    plsc.subcore_barrier()

    def body(j, carry):
        pltpu.sync_copy(w_hbm.at[wid * _NB + j], w_v)
        pltpu.sync_copy(w_v, acc.at[src_v.at[j]], add=True)
        return carry

    lax.fori_loop(0, _NB, body, 0)
    plsc.subcore_barrier()
    pltpu.sync_copy(acc.at[pl.ds(s * _NPT, _NPT)],
                    out_hbm.at[c, pl.ds(s * _NPT, _NPT)])


def _make_prop(F):
    nf = F // 16

    @functools.partial(
        pl.kernel,
        out_type=jax.ShapeDtypeStruct((_NC, _NA, F), jnp.float32),
        mesh=_MESH,
        compiler_params=pltpu.CompilerParams(needs_layout_passes=False, use_tc_tiling_on_sc=False),
        scratch_types=[
            pltpu.VMEM((_DR, 128), jnp.float32),
            pltpu.VMEM((_NB, _B), jnp.int32),
            pltpu.VMEM((_NB, _B), jnp.int32),
            pltpu.VMEM((_NB, _B), jnp.float32),
            pltpu.VMEM((_B,), jnp.float32),
            pltpu.VMEM((_B, F), jnp.float32),
            pltpu.VMEM((_B, F), jnp.float32),
            pltpu.VMEM_SHARED((_NA, F), jnp.float32),
            pltpu.SemaphoreType.DMA,
            pltpu.SemaphoreType.DMA,
            pltpu.SemaphoreType.DMA,
        ],
    )
    def _prop(h_hbm, dinv_hbm, src_hbm, dst_hbm, w_hbm, z_hbm, out_hbm,
              dinv_v, src_v, dst_v, w_v, norm_v, r0_v, r1_v, acc, g0, g1,
              sem_s):
        c = lax.axis_index("c")
        s = lax.axis_index("s")
        wid = s * _NC + c
        pltpu.sync_copy(dinv_hbm, dinv_v)
        pltpu.sync_copy(src_hbm.at[wid], src_v)
        pltpu.sync_copy(dst_hbm.at[wid], dst_v)
        pltpu.sync_copy(w_hbm.at[wid], w_v)
        pltpu.sync_copy(z_hbm.at[pl.ds(s * _NPT, _NPT)],
                        acc.at[pl.ds(s * _NPT, _NPT)])
        plsc.subcore_barrier()

        def norm_batch(j):
            for g in range(_B // 16):
                sl16 = pl.ds(g * 16, 16)
                s16 = src_v[j, sl16]
                d16 = dst_v[j, sl16]
                w16 = w_v[j, sl16]
                da = plsc.load_gather(
                    dinv_v,
                    [lax.shift_right_logical(s16, 7), lax.bitwise_and(s16, 127)])
                db = plsc.load_gather(
                    dinv_v,
                    [lax.shift_right_logical(d16, 7), lax.bitwise_and(d16, 127)])
                norm_v[sl16] = -(da * w16 * db)

        def scale(rows_ref):
            def body(q, carry):
                i0 = q * 4
                for k in range(4):
                    i = i0 + k
                    nv = plsc.load_gather(norm_v, [jnp.full((16,), i, jnp.int32)])
                    for f in range(nf):
                        slf = pl.ds(f * 16, 16)
                        rows_ref[i, slf] = rows_ref[i, slf] * nv
                return carry
            lax.fori_loop(0, _B // 4, body, 0)

        pltpu.async_copy(h_hbm.at[src_v.at[0]], r0_v, g0)

        def pair(jj, carry):
            j0 = 2 * jj
            j1 = j0 + 1
            j2 = jnp.minimum(j0 + 2, _NB - 1)
            d1 = pltpu.async_copy(h_hbm.at[src_v.at[j1]], r1_v, g1)
            norm_batch(j0)
            pltpu.make_async_copy(h_hbm.at[src_v.at[j0]], r0_v, g0).wait()
            scale(r0_v)
            sc0 = pltpu.async_copy(r0_v, acc.at[dst_v.at[j0]], sem_s, add=True)
            norm_batch(j1)
            d1.wait()
            scale(r1_v)
            sc0.wait()
            pltpu.async_copy(h_hbm.at[src_v.at[j2]], r0_v, g0)
            pltpu.sync_copy(r1_v, acc.at[dst_v.at[j1]], add=True)
            return carry

        lax.fori_loop(0, _NB // 2, pair, 0)
        pltpu.make_async_copy(h_hbm.at[src_v.at[_NB - 1]], r0_v, g0).wait()
        plsc.subcore_barrier()
        pltpu.sync_copy(acc.at[pl.ds(s * _NPT, _NPT)],
                        out_hbm.at[c, pl.ds(s * _NPT, _NPT)])

    return _prop


_prop64 = _make_prop(64)
_prop32 = _make_prop(32)


_P = lax.Precision.HIGHEST
_RB = 2000


def _tc_a_body(x_ref, w_ref, b_ref, degp_ref, h_ref, dinv_ref):
    deg = degp_ref[0, :, 0:1] + degp_ref[1, :, 0:1]
    safe = jnp.where(deg > 0, deg, 1.0)
    dinv_ref[...] = jnp.where(deg > 0, lax.rsqrt(safe), 0.0)
    h = jnp.dot(x_ref[...], w_ref[...], precision=_P) + b_ref[...][None, :]
    h_ref[...] = jnp.maximum(h, 0.0)


def _tc_b_body(p_ref, h_ref, cw_ref, tx1_ref, acc_ref):
    tx1 = p_ref[0] + p_ref[1]
    tx1_ref[...] = tx1
    acc_ref[...] = (jnp.dot(h_ref[...], cw_ref[0] - cw_ref[2], precision=_P)
                    + jnp.dot(tx1, cw_ref[1], precision=_P))


def _tc_c_body(q_ref, acc_ref, cw_ref, cb_ref, out_ref):
    tx2p = q_ref[0] + q_ref[1]
    out = acc_ref[...] + 2.0 * jnp.dot(tx2p, cw_ref[2], precision=_P)
    out_ref[...] = jnp.maximum(out + cb_ref[...][None, :], 0.0)


def _tc_head_body(q_ref, acc_ref, cw_ref, cb_ref, w2_ref, b2_ref, out_ref):
    tx2p = q_ref[0] + q_ref[1]
    h2 = acc_ref[...] + 2.0 * jnp.dot(tx2p, cw_ref[2], precision=_P)
    h2 = jnp.maximum(h2 + cb_ref[...][None, :], 0.0)
    logits = jnp.dot(h2, w2_ref[...], precision=_P) + b2_ref[...][None, :]
    m = jnp.max(logits, axis=1, keepdims=True)
    e = jnp.exp(logits - m)
    out_ref[...] = e / jnp.sum(e, axis=1, keepdims=True)


def _rows(shape):
    if len(shape) == 3:
        return pl.BlockSpec((2, _RB, shape[2]), lambda i: (0, i, 0))
    return pl.BlockSpec((_RB,) + shape[1:], lambda i: (i,) + (0,) * (len(shape) - 1))


def _full(shape):
    return pl.BlockSpec(shape, lambda i: (0,) * len(shape))


def _call(body, out_shapes, specs, *args):
    grid = (_N // _RB,)
    in_specs = [s(a.shape) for s, a in zip(specs, args)]
    out_specs = jax.tree.map(lambda o: _rows(o.shape), out_shapes,
                             is_leaf=lambda o: isinstance(o, jax.ShapeDtypeStruct))
    return pl.pallas_call(body, grid=grid, in_specs=in_specs,
                          out_specs=out_specs, out_shape=out_shapes)(*args)


def kernel(x, edge_index, edge_weight, W1, b1, c1_w, c1_b, c2_w, c2_b, W2, b2):
    pad = _EPAD - _E
    src = jnp.concatenate([edge_index[0], jnp.zeros((pad,), jnp.int32)])
    dst = jnp.concatenate([edge_index[1], jnp.zeros((pad,), jnp.int32)])
    w = jnp.concatenate([edge_weight, jnp.zeros((pad,), jnp.float32)])
    src2d = src.reshape(_NW, _NB, _B)
    dst2d = dst.reshape(_NW, _NB, _B)
    w2d = w.reshape(_NW, _NB, _B)
    w16 = jnp.broadcast_to(w[:, None], (_EPAD, 16)).reshape(_NW * _NB, _B, 16)
    z16 = jnp.zeros((_NA, 16), jnp.float32)
    z64 = jnp.zeros((_NA, 64), jnp.float32)
    z32 = jnp.zeros((_NA, 32), jnp.float32)

    degp = _deg_sc(src2d, w16, z16)
    h0, dinv1 = _call(
        _tc_a_body,
        (jax.ShapeDtypeStruct((_N, 64), jnp.float32),
         jax.ShapeDtypeStruct((_N, 1), jnp.float32)),
        (_rows, _full, _full, _rows),
        x, W1, b1, degp)
    dinv = jnp.pad(dinv1.reshape(_N), (0, _ND - _N)).reshape(_DR, 128)

    p1 = _prop64(h0, dinv, src2d, dst2d, w2d, z64)
    tx1, acc1 = _call(
        _tc_b_body,
        (jax.ShapeDtypeStruct((_N, 64), jnp.float32),
         jax.ShapeDtypeStruct((_N, 32), jnp.float32)),
        (_rows, _rows, _full),
        p1, h0, c1_w)
    q1 = _prop64(tx1, dinv, src2d, dst2d, w2d, z64)
    h1 = _call(_tc_c_body, jax.ShapeDtypeStruct((_N, 32), jnp.float32),
               (_rows, _rows, _full, _full), q1, acc1, c1_w, c1_b)

    p2 = _prop32(h1, dinv, src2d, dst2d, w2d, z32)
    tx1b, acc2 = _call(
        _tc_b_body,
        (jax.ShapeDtypeStruct((_N, 32), jnp.float32),
         jax.ShapeDtypeStruct((_N, 16), jnp.float32)),
        (_rows, _rows, _full),
        p2, h1, c2_w)
    q2 = _prop32(tx1b, dinv, src2d, dst2d, w2d, z32)
    probs = _call(_tc_head_body, jax.ShapeDtypeStruct((_N, 2), jnp.float32),
                  (_rows, _rows, _full, _full, _full, _full),
                  q2, acc2, c2_w, c2_b, W2, b2)
    return probs
